# baseline (speedup 1.0000x reference)
import functools

import jax
import jax.numpy as jnp
from jax import lax
from jax.experimental import pallas as pl
from jax.experimental.pallas import tpu as pltpu
from jax.experimental.pallas import tpu_sc as plsc

NUM_F = 9
VOCAB = 100
HIDDEN = 128
NWORKERS = 32
CHUNK = 32
XW = CHUNK * NUM_F
OW = CHUNK * HIDDEN
TAB_WORDS = NUM_F * VOCAB * HIDDEN


def _tree_sum9(vals):
    s01 = vals[0] + vals[1]
    s23 = vals[2] + vals[3]
    s45 = vals[4] + vals[5]
    s67 = vals[6] + vals[7]
    a = s01 + s23
    b = s45 + s67
    return (a + b) + vals[8]


def _pack_table(tab):
    r, c = tab.shape
    t = tab.astype(jnp.bfloat16).reshape(r, c // 32, 2, 16).transpose(0, 1, 3, 2)
    return lax.bitcast_convert_type(t, jnp.int32).reshape(-1)


def _body(rows_per_worker, last_rows, x_hbm, tab_hbm, out_hbm,
          tab_v, xb0, xb1, ob0, ob1, sx0, sx1, so0, so1):
    wid = lax.axis_index("s") * 2 + lax.axis_index("c")
    base_row = wid * rows_per_worker

    pltpu.sync_copy(tab_hbm, tab_v)

    my_rows = jnp.where(wid == NWORKERS - 1, last_rows, rows_per_worker)
    num_chunks = my_rows // CHUNK
    npairs = num_chunks // 2

    def x_slice(c):
        return x_hbm.at[pl.ds((base_row + c * CHUNK) * NUM_F, XW)]

    def o_slice(c):
        return out_hbm.at[pl.ds(base_row + c * CHUNK, CHUNK), :]

    def compute(xbuf, outbuf):
        @plsc.parallel_loop(0, CHUNK, 1, unroll=2)
        def row_body(r):
            xv = xbuf[pl.ds(r * NUM_F, 16)]
            bases = [xv[i] * (HIDDEN // 2) + i * (VOCAB * HIDDEN // 2)
                     for i in range(NUM_F)]
            for sb in range(HIDDEN // 32):
                loads = [plsc.bitcast(tab_v[pl.ds(bases[i] + sb * 16, 16)],
                                      jnp.bfloat16)
                         for i in range(NUM_F)]
                lo, hi = plsc.unpack(_tree_sum9(loads),
                                     format=plsc.PackFormat.INTERLEAVED)
                outbuf[r, pl.ds(sb * 32, 16)] = lo
                outbuf[r, pl.ds(sb * 32 + 16, 16)] = hi

    pltpu.async_copy(x_slice(0), xb0.at[pl.ds(0, XW)], sx0)

    def pair_body(p, _):
        ca = 2 * p
        cb = ca + 1
        pltpu.async_copy(x_slice(cb), xb1.at[pl.ds(0, XW)], sx1)

        pltpu.make_async_copy(x_slice(ca), xb0.at[pl.ds(0, XW)], sx0).wait()

        @pl.when(p > 0)
        def _():
            pltpu.make_async_copy(ob0, o_slice(ca), so0).wait()

        compute(xb0, ob0)
        pltpu.async_copy(ob0, o_slice(ca), so0)

        @pl.when(p < npairs - 1)
        def _():
            pltpu.async_copy(x_slice(ca + 2), xb0.at[pl.ds(0, XW)], sx0)

        pltpu.make_async_copy(x_slice(cb), xb1.at[pl.ds(0, XW)], sx1).wait()

        @pl.when(p > 0)
        def _():
            pltpu.make_async_copy(ob1, o_slice(cb), so1).wait()

        compute(xb1, ob1)
        pltpu.async_copy(ob1, o_slice(cb), so1)
        return 0

    lax.fori_loop(0, npairs, pair_body, 0)

    @pl.when(npairs > 0)
    def _():
        pltpu.make_async_copy(ob0, o_slice(0), so0).wait()
        pltpu.make_async_copy(ob1, o_slice(0), so1).wait()

    @pl.when(num_chunks % 2 == 1)
    def _():
        c = num_chunks - 1
        pltpu.sync_copy(x_slice(c), xb0.at[pl.ds(0, XW)])
        compute(xb0, ob0)
        pltpu.sync_copy(ob0, o_slice(c))


def kernel(x, tables):
    n = x.shape[0]
    n32 = ((n + CHUNK - 1) // CHUNK) * CHUNK
    if n32 != n:
        x = jnp.pad(x, ((0, n32 - n), (0, 0)))
    rows_per_worker = ((n32 + NWORKERS * CHUNK - 1) // (NWORKERS * CHUNK)) * CHUNK
    last_rows = n32 - (NWORKERS - 1) * rows_per_worker
    assert last_rows >= 0

    x_flat = x.reshape(-1)
    tab_flat = _pack_table(tables.reshape(NUM_F * VOCAB, HIDDEN))

    mesh = plsc.VectorSubcoreMesh(
        core_axis_name="c", subcore_axis_name="s", num_cores=2, num_subcores=16
    )
    run = pl.kernel(
        functools.partial(_body, rows_per_worker, last_rows),
        out_type=jax.ShapeDtypeStruct((n32, HIDDEN), jnp.float32),
        mesh=mesh,
        compiler_params=pltpu.CompilerParams(needs_layout_passes=False),
        scratch_types=[
            pltpu.VMEM((TAB_WORDS // 2,), jnp.int32),
            pltpu.VMEM((XW + 16,), jnp.int32),
            pltpu.VMEM((XW + 16,), jnp.int32),
            pltpu.VMEM((CHUNK, HIDDEN), jnp.float32),
            pltpu.VMEM((CHUNK, HIDDEN), jnp.float32),
            pltpu.SemaphoreType.DMA,
            pltpu.SemaphoreType.DMA,
            pltpu.SemaphoreType.DMA,
            pltpu.SemaphoreType.DMA,
        ],
    )
    out = run(x_flat, tab_flat)
    return out[:n] if n32 != n else out
